# baseline (speedup 1.0000x reference)
import functools

import jax
import jax.numpy as jnp
from jax import lax
from jax.experimental import pallas as pl
from jax.experimental.pallas import tpu as pltpu
from jax.experimental.pallas import tpu_sc as plsc


def _broadcast_emb(B, S, D, dtype):
    info = plsc.get_sparse_core_info()
    nw = info.num_cores * info.num_subcores
    rows_per_w = S // nw
    chunk = 32
    n_chunks = rows_per_w // chunk
    mesh = plsc.VectorSubcoreMesh(core_axis_name="c", subcore_axis_name="s")

    @functools.partial(
        pl.kernel,
        mesh=mesh,
        out_type=jax.ShapeDtypeStruct((B, S, D), dtype),
        scratch_types=[
            pltpu.VMEM((chunk, D), dtype),
            pltpu.VMEM((chunk, D), dtype),
            pltpu.SemaphoreType.DMA,
            pltpu.SemaphoreType.DMA,
            pltpu.SemaphoreType.DMA,
            pltpu.SemaphoreType.DMA,
        ],
    )
    def k(emb_hbm, out_hbm, buf0, buf1, rsem0, rsem1, wsem0, wsem1):
        wid = lax.axis_index("s") * info.num_cores + lax.axis_index("c")
        base = wid * rows_per_w
        bufs = (buf0, buf1)
        rsems = (rsem0, rsem1)
        wsems = (wsem0, wsem1)

        def start_read(i):
            r0 = base + i * chunk
            return pltpu.async_copy(
                emb_hbm.at[pl.ds(r0, chunk), :], bufs[i % 2], rsems[i % 2]
            )

        reads = {0: start_read(0)}
        writes = {}
        for i in range(n_chunks):
            c = i % 2
            reads.pop(i).wait()
            if i + 1 < n_chunks:
                for w in writes.pop(i - 1, ()):
                    w.wait()
                reads[i + 1] = start_read(i + 1)
            r0 = base + i * chunk
            writes[i] = [
                pltpu.async_copy(
                    bufs[c], out_hbm.at[b, pl.ds(r0, chunk), :], wsems[c]
                )
                for b in range(B)
            ]
        for ws in writes.values():
            for w in ws:
                w.wait()

    return k


def kernel(x, emb):
    B, S, D = x.shape
    return _broadcast_emb(B, S, D, emb.dtype)(emb)
